# Initial kernel scaffold; baseline (speedup 1.0000x reference)
import jax
import jax.numpy as jnp
from jax.experimental import pallas as pl


def kernel(input, W):
    raise NotImplementedError("write your pallas kernel here")


# speedup vs baseline: 2.0328x; 2.0328x over previous
import functools

import jax
import jax.numpy as jnp
from jax import lax
from jax.experimental import pallas as pl
from jax.experimental.pallas import tpu as pltpu
from jax.experimental.pallas import tpu_sc as plsc

TOPK = 2
E = 64
D = 2048
T = 8192
BT = 512
NB = T // BT

NC = 2
NS = 16
L = 16
EPS_PER_SUB = E // NS
CHUNKS = T // L
TPS = T // NS


def _gate_block(x_ref, wt_ref, idx1_ref, idx2_ref, g1_ref, g2_ref,
                laux_ref, ent_ref, cnt_ref, me_acc, cnt_acc, ent_acc):
    i = pl.program_id(0)

    @pl.when(i == 0)
    def _init():
        me_acc[...] = jnp.zeros_like(me_acc)
        cnt_acc[...] = jnp.zeros_like(cnt_acc)
        ent_acc[0, 0] = 0.0

    logits = jnp.dot(x_ref[...], wt_ref[...],
                     preferred_element_type=jnp.float32)
    m = jnp.max(logits, axis=1, keepdims=True)
    ex = jnp.exp(logits - m)
    gates = ex / jnp.sum(ex, axis=1, keepdims=True)

    eps = float(jnp.finfo(jnp.float32).eps)
    logp = jnp.log(jnp.clip(gates, eps, 1.0))
    ent_acc[0, 0] += -jnp.sum(gates * logp)

    iota = lax.broadcasted_iota(jnp.int32, (BT, E), 1)
    g1 = jnp.max(gates, axis=1, keepdims=True)
    idx1 = jnp.min(jnp.where(gates == g1, iota, E), axis=1, keepdims=True)
    gm = jnp.where(iota == idx1, -1.0, gates)
    g2 = jnp.max(gm, axis=1, keepdims=True)
    idx2 = jnp.min(jnp.where(gm == g2, iota, E), axis=1, keepdims=True)

    oh1 = (iota == idx1).astype(jnp.float32)
    me_acc[...] += jnp.sum(gates, axis=0, keepdims=True)
    cnt_acc[...] += jnp.sum(oh1, axis=0, keepdims=True)

    idx1_ref[...] = idx1
    idx2_ref[...] = idx2
    g1_ref[...] = g1
    g2_ref[...] = g2

    @pl.when(i == NB - 1)
    def _fini():
        laux = jnp.sum(me_acc[...] * cnt_acc[...]) * (E / T / T)
        laux_ref[...] = jnp.reshape(laux, (1, 1))
        ent_ref[...] = jnp.reshape(ent_acc[0, 0] * (1.0 / T), (1, 1))
        cnt_ref[...] = cnt_acc[...]


_gate_call = pl.pallas_call(
    _gate_block,
    grid=(NB,),
    in_specs=[
        pl.BlockSpec((BT, D), lambda i: (i, 0)),
        pl.BlockSpec((D, E), lambda i: (0, 0)),
    ],
    out_specs=[
        pl.BlockSpec((BT, 1), lambda i: (i, 0)),
        pl.BlockSpec((BT, 1), lambda i: (i, 0)),
        pl.BlockSpec((BT, 1), lambda i: (i, 0)),
        pl.BlockSpec((BT, 1), lambda i: (i, 0)),
        pl.BlockSpec((1, 1), lambda i: (0, 0)),
        pl.BlockSpec((1, 1), lambda i: (0, 0)),
        pl.BlockSpec((1, E), lambda i: (0, 0)),
    ],
    out_shape=[
        jax.ShapeDtypeStruct((T, 1), jnp.int32),
        jax.ShapeDtypeStruct((T, 1), jnp.int32),
        jax.ShapeDtypeStruct((T, 1), jnp.float32),
        jax.ShapeDtypeStruct((T, 1), jnp.float32),
        jax.ShapeDtypeStruct((1, 1), jnp.float32),
        jax.ShapeDtypeStruct((1, 1), jnp.float32),
        jax.ShapeDtypeStruct((1, E), jnp.float32),
    ],
    scratch_shapes=[
        pltpu.VMEM((1, E), jnp.float32),
        pltpu.VMEM((1, E), jnp.float32),
        pltpu.SMEM((1, 1), jnp.float32),
    ],
    compiler_params=pltpu.CompilerParams(
        dimension_semantics=("arbitrary",)),
)


def _sc_locations_body(idx_hbm, cnt1_hbm, loc_hbm,
                       idx_v, out_v, stage_v, res_v, cnt_v, shared):
    c = lax.axis_index("c")
    s = lax.axis_index("s")

    pltpu.sync_copy(idx_hbm.at[c], idx_v)
    pltpu.sync_copy(cnt1_hbm, cnt_v)

    iota16 = lax.iota(jnp.int32, L)
    zeros16 = jnp.zeros((L,), jnp.int32)

    my_chunk = zeros16
    for k in range(E // L):
        ck = cnt_v[pl.ds(k * L, L)]
        my_chunk = my_chunk + jnp.where((s // (L // EPS_PER_SUB)) == k, ck, 0)
    is_core1 = (c == 1).astype(jnp.int32)
    lane0 = (s % (L // EPS_PER_SUB)) * EPS_PER_SUB
    seeds = []
    for j in range(EPS_PER_SUB):
        cj = jnp.sum(jnp.where(iota16 == lane0 + j, my_chunk, 0))
        seeds.append(cj * is_core1)

    def _zero(i, carry):
        out_v[pl.ds(i * L, L)] = zeros16
        return carry

    lax.fori_loop(0, CHUNKS, _zero, 0)

    e_base = s * EPS_PER_SUB

    def _scan(i, counters):
        v = idx_v[pl.ds(i * L, L)]
        pos = iota16 + i * L
        new = []
        for j in range(EPS_PER_SUB):
            mask = v == (e_base + j)
            mi = jnp.where(mask, 1, 0)
            incl = plsc.cumsum(mi)
            loc = incl - mi + counters[j]
            plsc.store_scatter(out_v, [pos], loc, mask=mask)
            new.append(counters[j] + jnp.sum(mi))
        return tuple(new)

    lax.fori_loop(0, CHUNKS, _scan, tuple(seeds))

    pltpu.sync_copy(out_v, shared.at[s])
    plsc.subcore_barrier()
    pltpu.sync_copy(shared.at[:, pl.ds(s * TPS, TPS)], stage_v)

    def _reduce(q, carry):
        acc = zeros16
        for r in range(NS):
            acc = acc + stage_v[r, pl.ds(q * L, L)]
        res_v[pl.ds(q * L, L)] = acc
        return carry

    lax.fori_loop(0, TPS // L, _reduce, 0)

    pltpu.sync_copy(res_v, loc_hbm.at[c, pl.ds(s * TPS, TPS)])


@functools.cache
def _sc_locations():
    return pl.kernel(
        _sc_locations_body,
        out_type=jax.ShapeDtypeStruct((TOPK, T), jnp.int32),
        mesh=plsc.VectorSubcoreMesh(
            core_axis_name="c", subcore_axis_name="s",
            num_cores=NC, num_subcores=NS),
        scratch_types=[
            pltpu.VMEM((T,), jnp.int32),
            pltpu.VMEM((T,), jnp.int32),
            pltpu.VMEM((NS, TPS), jnp.int32),
            pltpu.VMEM((TPS,), jnp.int32),
            pltpu.VMEM((E,), jnp.int32),
            pltpu.VMEM_SHARED((NS, T), jnp.int32),
        ],
        compiler_params=pltpu.CompilerParams(needs_layout_passes=False),
    )


def kernel(input, W):
    wt = W.T
    idx1, idx2, g1, g2, laux, ent, cnt = _gate_call(input, wt)
    i1 = idx1.reshape(T)
    i2 = idx2.reshape(T)
    cnt1 = cnt.reshape(E).astype(jnp.int32)
    indices_s = jnp.stack([i1, i2], axis=0)
    locations_s = _sc_locations()(indices_s, cnt1)
    gates_s = jnp.stack([g1.reshape(T), g2.reshape(T)], axis=0)
    return (laux[0, 0], ent[0, 0], indices_s, locations_s, gates_s)
